# baseline (speedup 1.0000x reference)
import jax
import jax.numpy as jnp
from jax import lax
from jax.experimental import pallas as pl
from jax.experimental.pallas import tpu as pltpu
from jax.experimental.pallas import tpu_sc as plsc

NC = 2
NS = 16
NW = NC * NS
L = 16

B, S, D = 2, 8192, 4096
NWIN = 64
OUT_TOTAL = B * S * NWIN
OUT_W = OUT_TOTAL // NW
BLK = 128
NBLK = OUT_W // BLK
HALF = 8
NROUND = NBLK // HALF


def _sc_window_select(xt, w_hbm, out, w_v, idx_v, obuf, sem_a, sem_b):
    wid = lax.axis_index("s") * NC + lax.axis_index("c")
    wbase = wid * OUT_W

    pltpu.sync_copy(w_hbm, w_v)

    wvec = [w_v[pl.ds(t * L, L)] for t in range(4)]
    xbase = [v + wbase * 64 for v in wvec]

    def fill(blk, carry):
        b0 = blk * 8192
        for t in range(BLK // L):
            idx_v[blk, pl.ds(t * L, L)] = xbase[t % 4] + (b0 + (t // 4) * 4096)
        return carry

    lax.fori_loop(0, NBLK, fill, 0)

    def dma(k, i, sem):
        blk = k * HALF + i
        return pltpu.make_async_copy(
            xt.at[idx_v.at[blk]], obuf.at[pl.ds(blk * BLK, BLK)], sem
        )

    def fire(k, sem):
        for i in range(HALF):
            dma(k, i, sem).start()

    def drain(k, sem):
        for i in range(HALF):
            dma(k, i, sem).wait()

    fire(0, sem_a)

    def super_round(r, carry):
        k0 = 2 * r
        fire(k0 + 1, sem_b)
        drain(k0, sem_a)

        @pl.when(r < NROUND // 2 - 1)
        def _():
            fire(k0 + 2, sem_a)

        drain(k0 + 1, sem_b)
        return carry

    lax.fori_loop(0, NROUND // 2, super_round, 0)
    pltpu.sync_copy(obuf, out.at[pl.ds(wbase, OUT_W)])


@jax.jit
def kernel(x, w):
    xt = x.reshape(B * S * D)
    w32 = w.astype(jnp.int32)
    run = pl.kernel(
        _sc_window_select,
        out_type=jax.ShapeDtypeStruct((OUT_TOTAL,), jnp.float32),
        mesh=plsc.VectorSubcoreMesh(core_axis_name="c", subcore_axis_name="s"),
        scratch_types=[
            pltpu.VMEM((NWIN,), jnp.int32),
            pltpu.VMEM((NBLK, BLK), jnp.int32),
            pltpu.VMEM((OUT_W,), jnp.float32),
            pltpu.SemaphoreType.DMA,
            pltpu.SemaphoreType.DMA,
        ],
    )
    out = run(xt, w32)
    return out.reshape(B, S, NWIN)
